# baseline (speedup 1.0000x reference)
import functools

import jax
import jax.numpy as jnp
from jax.experimental import pallas as pl

H = 8
BQ = 256


def _qkv_proj_kernel(x_ref, w_ref, b_ref, o_ref):
    o_ref[:] = jnp.dot(x_ref[:], w_ref[:], preferred_element_type=jnp.float32) + b_ref[:]


def _attn_kernel(scale, bidx_ref, q_ref, k_ref, v_ref, o_ref):
    i = pl.program_id(1)
    q = q_ref[0]
    k = k_ref[0]
    v = v_ref[0]
    s = jax.lax.dot_general(
        q, k, (((1,), (1,)), ((), ())), preferred_element_type=jnp.float32
    ) * scale
    bq = bidx_ref[0, pl.ds(i * BQ, BQ)]
    mask = bq[:, None] == bidx_ref[0][None, :]
    s = jnp.where(mask, s, -1e9)
    m = jnp.max(s, axis=-1, keepdims=True)
    p = jnp.exp(s - m)
    p = p / jnp.sum(p, axis=-1, keepdims=True)
    o_ref[0] = jnp.dot(p, v, preferred_element_type=jnp.float32)


def kernel(feats, coords, W_qkv, b_qkv, W_out, b_out):
    N, C = feats.shape
    D = C // H
    scale = D ** -0.5

    qkv = pl.pallas_call(
        _qkv_proj_kernel,
        grid=(N // BQ,),
        in_specs=[
            pl.BlockSpec((BQ, C), lambda i: (i, 0)),
            pl.BlockSpec((C, 3 * C), lambda i: (0, 0)),
            pl.BlockSpec((1, 3 * C), lambda i: (0, 0)),
        ],
        out_specs=pl.BlockSpec((BQ, 3 * C), lambda i: (i, 0)),
        out_shape=jax.ShapeDtypeStruct((N, 3 * C), jnp.float32),
    )(feats, W_qkv.T, b_qkv[None, :])

    qkv = qkv.reshape(N, 3, H, D)
    q = qkv[:, 0].transpose(1, 0, 2)
    k = qkv[:, 1].transpose(1, 0, 2)
    v = qkv[:, 2].transpose(1, 0, 2)
    bidx = coords[:, 0].astype(jnp.int32)[None, :]

    attn_out = pl.pallas_call(
        functools.partial(_attn_kernel, scale),
        grid=(H, N // BQ),
        in_specs=[
            pl.BlockSpec((1, N), lambda h, i: (0, 0)),
            pl.BlockSpec((1, BQ, D), lambda h, i: (h, i, 0)),
            pl.BlockSpec((1, N, D), lambda h, i: (h, 0, 0)),
            pl.BlockSpec((1, N, D), lambda h, i: (h, 0, 0)),
        ],
        out_specs=pl.BlockSpec((1, BQ, D), lambda h, i: (h, i, 0)),
        out_shape=jax.ShapeDtypeStruct((H, N, D), jnp.float32),
    )(bidx, q, k, v)

    attn_out = attn_out.transpose(1, 0, 2).reshape(N, C)

    out = pl.pallas_call(
        _qkv_proj_kernel,
        grid=(N // BQ,),
        in_specs=[
            pl.BlockSpec((BQ, C), lambda i: (i, 0)),
            pl.BlockSpec((C, C), lambda i: (0, 0)),
            pl.BlockSpec((1, C), lambda i: (0, 0)),
        ],
        out_specs=pl.BlockSpec((BQ, C), lambda i: (i, 0)),
        out_shape=jax.ShapeDtypeStruct((N, C), jnp.float32),
    )(attn_out, W_out.T, b_out[None, :])

    return out
